# baseline (speedup 1.0000x reference)
import functools

import jax
import jax.numpy as jnp
from jax import lax
from jax.experimental import pallas as pl
from jax.experimental.pallas import tpu as pltpu
from jax.experimental.pallas import tpu_sc as plsc

N = 10000
E = 320000
D = 128
NW = 32
CHUNK = 80
NCH = E // NW // CHUNK
NBUF = 3
KB = 25
NBLK = NCH // KB
ROWS_PER_TILE = N // 16

_mesh = plsc.VectorSubcoreMesh(core_axis_name="c", subcore_axis_name="s")
_sc_params = pltpu.CompilerParams(use_tc_tiling_on_sc=False)


def _sc_agg_body(with_count, *refs):
    if with_count:
        (x_hbm, src_hbm, dst_hbm, z128_hbm, z16_hbm, ones_hbm,
         pacc_hbm, pcnt_hbm, sblk, dblk, rows, ones_v, acc, cnt,
         sem_ib, sem_g, sem_s, sem_c) = refs
    else:
        (x_hbm, src_hbm, dst_hbm, z128_hbm,
         pacc_hbm, sblk, dblk, rows, acc, sem_ib, sem_g, sem_s) = refs
        sem_c = None
    c = lax.axis_index("c")
    s = lax.axis_index("s")
    wid = s * 2 + c
    sl = pl.ds(s * ROWS_PER_TILE, ROWS_PER_TILE)
    pltpu.sync_copy(z128_hbm, acc.at[sl])
    if with_count:
        pltpu.sync_copy(z16_hbm, cnt.at[sl])
        pltpu.sync_copy(ones_hbm, ones_v)
    plsc.subcore_barrier()

    def start_blk(B, t):
        pltpu.async_copy(src_hbm.at[wid, B], sblk.at[t], sem_ib.at[t])
        pltpu.async_copy(dst_hbm.at[wid, B], dblk.at[t], sem_ib.at[t])

    def wait_blk(t):
        pltpu.make_async_copy(src_hbm.at[0, 0], sblk.at[t], sem_ib.at[t]).wait()
        pltpu.make_async_copy(dst_hbm.at[0, 0], dblk.at[t], sem_ib.at[t]).wait()

    def start_gather(j):
        t = lax.rem(lax.div(j, KB), 2)
        k = lax.rem(j, KB)
        b = lax.rem(j, NBUF)
        pltpu.async_copy(x_hbm.at[sblk.at[t, k]], rows.at[b], sem_g.at[b])

    def wait_gather(b):
        pltpu.make_async_copy(x_hbm.at[pl.ds(0, CHUNK)], rows.at[b],
                              sem_g.at[b]).wait()

    def start_scatter(j):
        t = lax.rem(lax.div(j, KB), 2)
        k = lax.rem(j, KB)
        b = lax.rem(j, NBUF)
        pltpu.async_copy(rows.at[b], acc.at[dblk.at[t, k]], sem_s.at[b],
                         add=True)
        if with_count:
            pltpu.async_copy(ones_v, cnt.at[dblk.at[t, k]], sem_c.at[b],
                             add=True)

    def wait_scatter(b):
        pltpu.make_async_copy(rows.at[b], acc.at[pl.ds(0, CHUNK)],
                              sem_s.at[b]).wait()
        if with_count:
            pltpu.make_async_copy(ones_v, cnt.at[pl.ds(0, CHUNK)],
                                  sem_c.at[b]).wait()

    start_blk(0, 0)
    start_blk(1, 1)
    wait_blk(0)
    start_gather(0)

    def body(j, carry):
        b = lax.rem(j, NBUF)

        @pl.when(jnp.logical_and(j + 1 < NCH, lax.rem(j + 1, KB) == 0))
        def _():
            wait_blk(lax.rem(lax.div(j + 1, KB), 2))

        @pl.when(j + 1 < NCH)
        def _():
            start_gather(j + 1)

        wait_gather(b)
        start_scatter(j)

        @pl.when(j >= 1)
        def _():
            wait_scatter(lax.rem(j - 1, NBUF))

        nxt = lax.div(j, KB) + 1

        @pl.when(jnp.logical_and(lax.rem(j, KB) == 1,
                                 jnp.logical_and(j > KB, nxt < NBLK)))
        def _():
            start_blk(nxt, lax.rem(nxt, 2))

        return carry

    lax.fori_loop(0, NCH, body, 0)
    wait_scatter(lax.rem(NCH - 1, NBUF))
    plsc.subcore_barrier()
    pltpu.sync_copy(acc.at[sl], pacc_hbm.at[c, sl])
    if with_count:
        pltpu.sync_copy(cnt.at[sl], pcnt_hbm.at[c, sl])


_sc_agg_count = functools.partial(
    pl.kernel,
    out_type=(jax.ShapeDtypeStruct((2, N, D), jnp.float32),
              jax.ShapeDtypeStruct((2, N, 16), jnp.float32)),
    scratch_types=[
        pltpu.VMEM((2, KB, CHUNK), jnp.int32),
        pltpu.VMEM((2, KB, CHUNK), jnp.int32),
        pltpu.VMEM((NBUF, CHUNK, D), jnp.float32),
        pltpu.VMEM((CHUNK, 16), jnp.float32),
        pltpu.VMEM_SHARED((N, D), jnp.float32),
        pltpu.VMEM_SHARED((N, 16), jnp.float32),
        pltpu.SemaphoreType.DMA((2,)),
        pltpu.SemaphoreType.DMA((NBUF,)),
        pltpu.SemaphoreType.DMA((NBUF,)),
        pltpu.SemaphoreType.DMA((NBUF,)),
    ],
    mesh=_mesh,
    compiler_params=_sc_params,
)(functools.partial(_sc_agg_body, True))


_sc_agg = functools.partial(
    pl.kernel,
    out_type=jax.ShapeDtypeStruct((2, N, D), jnp.float32),
    scratch_types=[
        pltpu.VMEM((2, KB, CHUNK), jnp.int32),
        pltpu.VMEM((2, KB, CHUNK), jnp.int32),
        pltpu.VMEM((NBUF, CHUNK, D), jnp.float32),
        pltpu.VMEM_SHARED((N, D), jnp.float32),
        pltpu.SemaphoreType.DMA((2,)),
        pltpu.SemaphoreType.DMA((NBUF,)),
        pltpu.SemaphoreType.DMA((NBUF,)),
    ],
    mesh=_mesh,
    compiler_params=_sc_params,
)(functools.partial(_sc_agg_body, False))


_BN = 1000
_ROW_SPECS = [
    pl.BlockSpec((2, _BN, D), lambda i: (0, i, 0)),
    pl.BlockSpec((2, _BN, 16), lambda i: (0, i, 0)),
    pl.BlockSpec((_BN, D), lambda i: (i, 0)),
]
_W_SPEC = pl.BlockSpec((D, D), lambda i: (0, 0))
_B_SPEC = pl.BlockSpec((1, D), lambda i: (0, 0))
_OUT_SPEC = pl.BlockSpec((_BN, D), lambda i: (i, 0))
_OUT_SHAPE = jax.ShapeDtypeStruct((N, D), jnp.float32)


def _agg_of(pa_ref, pc_ref):
    asum = pa_ref[0] + pa_ref[1]
    cnt = pc_ref[0, :, 0:1] + pc_ref[1, :, 0:1]
    return asum / jnp.maximum(cnt, 1.0)


def _tc_pre_body(x_ref, wr_ref, b_ref, o_ref):
    o_ref[...] = (jnp.dot(x_ref[...], wr_ref[...],
                          preferred_element_type=jnp.float32) + b_ref[...])


def _tc_mid_body(pa_ref, pc_ref, xr_ref, wl_ref, w2r_ref, b2_ref,
                 oh_ref, ohr_ref):
    agg = _agg_of(pa_ref, pc_ref)
    h = jnp.maximum(
        jnp.dot(agg, wl_ref[...], preferred_element_type=jnp.float32)
        + xr_ref[...], 0.0)
    oh_ref[...] = h
    ohr_ref[...] = (jnp.dot(h, w2r_ref[...],
                            preferred_element_type=jnp.float32) + b2_ref[...])


def _tc_fin_body(pa_ref, pc_ref, hr_ref, wl_ref, o_ref):
    agg = _agg_of(pa_ref, pc_ref)
    o_ref[...] = (jnp.dot(agg, wl_ref[...],
                          preferred_element_type=jnp.float32) + hr_ref[...])


def _tc_pre(x, W_r, b):
    return pl.pallas_call(
        _tc_pre_body, grid=(N // _BN,),
        in_specs=[_ROW_SPECS[2], _W_SPEC, _B_SPEC],
        out_specs=_OUT_SPEC, out_shape=_OUT_SHAPE,
    )(x, W_r, b.reshape(1, D))


def _tc_mid(pacc, pcnt, xr, W1_l, W2_r, b2):
    return pl.pallas_call(
        _tc_mid_body, grid=(N // _BN,),
        in_specs=[*_ROW_SPECS, _W_SPEC, _W_SPEC, _B_SPEC],
        out_specs=(_OUT_SPEC, _OUT_SPEC),
        out_shape=(_OUT_SHAPE, _OUT_SHAPE),
    )(pacc, pcnt, xr, W1_l, W2_r, b2.reshape(1, D))


def _tc_fin(pacc, pcnt, hr, W2_l):
    return pl.pallas_call(
        _tc_fin_body, grid=(N // _BN,),
        in_specs=[*_ROW_SPECS, _W_SPEC],
        out_specs=_OUT_SPEC, out_shape=_OUT_SHAPE,
    )(pacc, pcnt, hr, W2_l)


def kernel(x, edge_index, W1_l, b1_l, W1_r, W2_l, b2_l, W2_r):
    ei = edge_index.astype(jnp.int32)
    src3 = ei[0].reshape(NW, NBLK, KB, CHUNK)
    dst3 = ei[1].reshape(NW, NBLK, KB, CHUNK)
    z128 = jnp.zeros((ROWS_PER_TILE, D), jnp.float32)
    z16 = jnp.zeros((ROWS_PER_TILE, 16), jnp.float32)
    ones16 = jnp.ones((CHUNK, 16), jnp.float32)

    xr1 = _tc_pre(x, W1_r, b1_l)
    pacc1, pcnt = _sc_agg_count(x, src3, dst3, z128, z16, ones16)
    h, hr2 = _tc_mid(pacc1, pcnt, xr1, W1_l, W2_r, b2_l)
    pacc2 = _sc_agg(h, src3, dst3, z128)
    return _tc_fin(pacc2, pcnt, hr2, W2_l)


# speedup vs baseline: 1.0080x; 1.0080x over previous
import functools

import jax
import jax.numpy as jnp
from jax import lax
from jax.experimental import pallas as pl
from jax.experimental.pallas import tpu as pltpu
from jax.experimental.pallas import tpu_sc as plsc

N = 10000
E = 320000
D = 128
NW = 32
CHUNK = 80
NCH = E // NW // CHUNK
NBUF = 3
KB = 25
NBLK = NCH // KB
ROWS_PER_TILE = N // 16

_mesh = plsc.VectorSubcoreMesh(core_axis_name="c", subcore_axis_name="s")
_sc_params = pltpu.CompilerParams(use_tc_tiling_on_sc=False)


def _sc_agg_body(with_count, *refs):
    if with_count:
        (x_hbm, src_hbm, dst_hbm, z128_hbm, z16_hbm, ones_hbm,
         pacc_hbm, pcnt_hbm, sblk, dblk, rows, ones_v, acc, cnt,
         sem_ib, sem_g, sem_s, sem_c) = refs
    else:
        (x_hbm, src_hbm, dst_hbm, z128_hbm,
         pacc_hbm, sblk, dblk, rows, acc, sem_ib, sem_g, sem_s) = refs
        sem_c = None
    c = lax.axis_index("c")
    s = lax.axis_index("s")
    wid = s * 2 + c
    sl = pl.ds(s * ROWS_PER_TILE, ROWS_PER_TILE)
    pltpu.sync_copy(z128_hbm, acc.at[sl])
    if with_count:
        pltpu.sync_copy(z16_hbm, cnt.at[sl])
        pltpu.sync_copy(ones_hbm, ones_v)
    plsc.subcore_barrier()

    def start_blk(B, t):
        pltpu.async_copy(src_hbm.at[wid, B], sblk.at[t], sem_ib.at[t])
        pltpu.async_copy(dst_hbm.at[wid, B], dblk.at[t], sem_ib.at[t])

    def wait_blk(t):
        pltpu.make_async_copy(src_hbm.at[0, 0], sblk.at[t], sem_ib.at[t]).wait()
        pltpu.make_async_copy(dst_hbm.at[0, 0], dblk.at[t], sem_ib.at[t]).wait()

    def start_gather(j):
        t = lax.rem(lax.div(j, KB), 2)
        k = lax.rem(j, KB)
        b = lax.rem(j, NBUF)
        pltpu.async_copy(x_hbm.at[sblk.at[t, k]], rows.at[b], sem_g.at[b])

    def wait_gather(b):
        pltpu.make_async_copy(x_hbm.at[pl.ds(0, CHUNK)], rows.at[b],
                              sem_g.at[b]).wait()

    def start_scatter(j):
        t = lax.rem(lax.div(j, KB), 2)
        k = lax.rem(j, KB)
        b = lax.rem(j, NBUF)
        pltpu.async_copy(rows.at[b], acc.at[dblk.at[t, k]], sem_s.at[b],
                         add=True)
        if with_count:
            pltpu.async_copy(ones_v, cnt.at[dblk.at[t, k]], sem_c.at[b],
                             add=True)

    def wait_scatter(b):
        pltpu.make_async_copy(rows.at[b], acc.at[pl.ds(0, CHUNK)],
                              sem_s.at[b]).wait()
        if with_count:
            pltpu.make_async_copy(ones_v, cnt.at[pl.ds(0, CHUNK)],
                                  sem_c.at[b]).wait()

    start_blk(0, 0)
    start_blk(1, 1)
    wait_blk(0)
    start_gather(0)

    def body(j, carry):
        b = lax.rem(j, NBUF)

        @pl.when(jnp.logical_and(j + 1 < NCH, lax.rem(j + 1, KB) == 0))
        def _():
            wait_blk(lax.rem(lax.div(j + 1, KB), 2))

        @pl.when(j + 1 < NCH)
        def _():
            start_gather(j + 1)

        wait_gather(b)
        start_scatter(j)

        @pl.when(j >= 1)
        def _():
            wait_scatter(lax.rem(j - 1, NBUF))

        nxt = lax.div(j, KB) + 1

        @pl.when(jnp.logical_and(lax.rem(j, KB) == 1,
                                 jnp.logical_and(j > KB, nxt < NBLK)))
        def _():
            start_blk(nxt, lax.rem(nxt, 2))

        return carry

    lax.fori_loop(0, NCH, body, 0)
    wait_scatter(lax.rem(NCH - 1, NBUF))
    plsc.subcore_barrier()
    pltpu.sync_copy(acc.at[sl], pacc_hbm.at[c, sl])
    if with_count:
        pltpu.sync_copy(cnt.at[sl], pcnt_hbm.at[c, sl])


_sc_agg_count = functools.partial(
    pl.kernel,
    out_type=(jax.ShapeDtypeStruct((2, N, D), jnp.float32),
              jax.ShapeDtypeStruct((2, N, 16), jnp.float32)),
    scratch_types=[
        pltpu.VMEM((2, KB, CHUNK), jnp.int32),
        pltpu.VMEM((2, KB, CHUNK), jnp.int32),
        pltpu.VMEM((NBUF, CHUNK, D), jnp.float32),
        pltpu.VMEM((CHUNK, 16), jnp.float32),
        pltpu.VMEM_SHARED((N, D), jnp.float32),
        pltpu.VMEM_SHARED((N, 16), jnp.float32),
        pltpu.SemaphoreType.DMA((2,)),
        pltpu.SemaphoreType.DMA((NBUF,)),
        pltpu.SemaphoreType.DMA((NBUF,)),
        pltpu.SemaphoreType.DMA((NBUF,)),
    ],
    mesh=_mesh,
    compiler_params=_sc_params,
)(functools.partial(_sc_agg_body, True))


_sc_agg = functools.partial(
    pl.kernel,
    out_type=jax.ShapeDtypeStruct((2, N, D), jnp.float32),
    scratch_types=[
        pltpu.VMEM((2, KB, CHUNK), jnp.int32),
        pltpu.VMEM((2, KB, CHUNK), jnp.int32),
        pltpu.VMEM((NBUF, CHUNK, D), jnp.float32),
        pltpu.VMEM_SHARED((N, D), jnp.float32),
        pltpu.SemaphoreType.DMA((2,)),
        pltpu.SemaphoreType.DMA((NBUF,)),
        pltpu.SemaphoreType.DMA((NBUF,)),
    ],
    mesh=_mesh,
    compiler_params=_sc_params,
)(functools.partial(_sc_agg_body, False))


_BN = 1000
_ROW_SPECS = [
    pl.BlockSpec((2, _BN, D), lambda i: (0, i, 0)),
    pl.BlockSpec((2, _BN, 16), lambda i: (0, i, 0)),
    pl.BlockSpec((_BN, D), lambda i: (i, 0)),
]
_W_SPEC = pl.BlockSpec((D, D), lambda i: (0, 0))
_B_SPEC = pl.BlockSpec((1, D), lambda i: (0, 0))
_OUT_SPEC = pl.BlockSpec((_BN, D), lambda i: (i, 0))
_OUT_SHAPE = jax.ShapeDtypeStruct((N, D), jnp.float32)


def _agg_of(pa_ref, pc_ref):
    asum = pa_ref[0] + pa_ref[1]
    cnt = pc_ref[0, :, 0:1] + pc_ref[1, :, 0:1]
    return asum / jnp.maximum(cnt, 1.0)


def _tc_mid_body(pa_ref, pc_ref, x_ref, wl_ref, b1_ref, w1r_ref, w2r_ref,
                 b2_ref, oh_ref, ohr_ref):
    agg = _agg_of(pa_ref, pc_ref)
    h = jnp.maximum(
        jnp.dot(agg, wl_ref[...], preferred_element_type=jnp.float32)
        + b1_ref[...]
        + jnp.dot(x_ref[...], w1r_ref[...], preferred_element_type=jnp.float32),
        0.0)
    oh_ref[...] = h
    ohr_ref[...] = (jnp.dot(h, w2r_ref[...],
                            preferred_element_type=jnp.float32) + b2_ref[...])


def _tc_fin_body(pa_ref, pc_ref, hr_ref, wl_ref, o_ref):
    agg = _agg_of(pa_ref, pc_ref)
    o_ref[...] = (jnp.dot(agg, wl_ref[...],
                          preferred_element_type=jnp.float32) + hr_ref[...])


def _tc_mid(pacc, pcnt, x, W1_l, b1, W1_r, W2_r, b2):
    return pl.pallas_call(
        _tc_mid_body, grid=(N // _BN,),
        in_specs=[*_ROW_SPECS, _W_SPEC, _B_SPEC, _W_SPEC, _W_SPEC, _B_SPEC],
        out_specs=(_OUT_SPEC, _OUT_SPEC),
        out_shape=(_OUT_SHAPE, _OUT_SHAPE),
    )(pacc, pcnt, x, W1_l, b1.reshape(1, D), W1_r, W2_r, b2.reshape(1, D))


def _tc_fin(pacc, pcnt, hr, W2_l):
    return pl.pallas_call(
        _tc_fin_body, grid=(N // _BN,),
        in_specs=[*_ROW_SPECS, _W_SPEC],
        out_specs=_OUT_SPEC, out_shape=_OUT_SHAPE,
    )(pacc, pcnt, hr, W2_l)


def kernel(x, edge_index, W1_l, b1_l, W1_r, W2_l, b2_l, W2_r):
    ei = edge_index.astype(jnp.int32)
    src3 = ei[0].reshape(NW, NBLK, KB, CHUNK)
    dst3 = ei[1].reshape(NW, NBLK, KB, CHUNK)
    z128 = jnp.zeros((ROWS_PER_TILE, D), jnp.float32)
    z16 = jnp.zeros((ROWS_PER_TILE, 16), jnp.float32)
    ones16 = jnp.ones((CHUNK, 16), jnp.float32)

    pacc1, pcnt = _sc_agg_count(x, src3, dst3, z128, z16, ones16)
    h, hr2 = _tc_mid(pacc1, pcnt, x, W1_l, b1_l, W1_r, W2_r, b2_l)
    pacc2 = _sc_agg(h, src3, dst3, z128)
    return _tc_fin(pacc2, pcnt, hr2, W2_l)


# speedup vs baseline: 1.0188x; 1.0107x over previous
import functools

import jax
import jax.numpy as jnp
from jax import lax
from jax.experimental import pallas as pl
from jax.experimental.pallas import tpu as pltpu
from jax.experimental.pallas import tpu_sc as plsc

N = 10000
E = 320000
D = 128
NW = 32
CHUNK = 80
NCH = E // NW // CHUNK
NBUF = 3
KB = 25
NBLK = NCH // KB
ROWS_PER_TILE = N // 16

_mesh = plsc.VectorSubcoreMesh(core_axis_name="c", subcore_axis_name="s")
_sc_params = pltpu.CompilerParams(use_tc_tiling_on_sc=False)


def _sc_agg_body(with_count, *refs):
    if with_count:
        (x_hbm, src_hbm, dst_hbm, z128_hbm, z16_hbm, ones_hbm,
         pacc_hbm, pcnt_hbm, sblk, dblk, rows, ones_v, acc, cnt,
         sem_ib, sem_g, sem_s, sem_c) = refs
    else:
        (x_hbm, src_hbm, dst_hbm, z128_hbm,
         pacc_hbm, sblk, dblk, rows, acc, sem_ib, sem_g, sem_s) = refs
        sem_c = None
    c = lax.axis_index("c")
    s = lax.axis_index("s")
    wid = s * 2 + c
    sl = pl.ds(s * ROWS_PER_TILE, ROWS_PER_TILE)
    pltpu.sync_copy(z128_hbm, acc.at[sl])
    if with_count:
        pltpu.sync_copy(z16_hbm, cnt.at[sl])
        pltpu.sync_copy(ones_hbm, ones_v)
    plsc.subcore_barrier()

    def start_blk(B, t):
        pltpu.async_copy(src_hbm.at[wid, B], sblk.at[t], sem_ib.at[t])
        pltpu.async_copy(dst_hbm.at[wid, B], dblk.at[t], sem_ib.at[t])

    def wait_blk(t):
        pltpu.make_async_copy(src_hbm.at[0, 0], sblk.at[t], sem_ib.at[t]).wait()
        pltpu.make_async_copy(dst_hbm.at[0, 0], dblk.at[t], sem_ib.at[t]).wait()

    def start_gather(j):
        t = lax.rem(lax.div(j, KB), 2)
        k = lax.rem(j, KB)
        b = lax.rem(j, NBUF)
        pltpu.async_copy(x_hbm.at[sblk.at[t, k]], rows.at[b], sem_g.at[b])

    def wait_gather(b):
        pltpu.make_async_copy(x_hbm.at[pl.ds(0, CHUNK)], rows.at[b],
                              sem_g.at[b]).wait()

    def start_scatter(j):
        t = lax.rem(lax.div(j, KB), 2)
        k = lax.rem(j, KB)
        b = lax.rem(j, NBUF)
        pltpu.async_copy(rows.at[b], acc.at[dblk.at[t, k]], sem_s.at[b],
                         add=True)
        if with_count:
            pltpu.async_copy(ones_v, cnt.at[dblk.at[t, k]], sem_c.at[b],
                             add=True)

    def wait_scatter(b):
        pltpu.make_async_copy(rows.at[b], acc.at[pl.ds(0, CHUNK)],
                              sem_s.at[b]).wait()
        if with_count:
            pltpu.make_async_copy(ones_v, cnt.at[pl.ds(0, CHUNK)],
                                  sem_c.at[b]).wait()

    start_blk(0, 0)
    start_blk(1, 1)
    wait_blk(0)
    start_gather(0)
    start_gather(1)

    def body(j, carry):
        b = lax.rem(j, NBUF)

        wait_gather(b)
        start_scatter(j)

        @pl.when(j >= 1)
        def _():
            wait_scatter(lax.rem(j - 1, NBUF))

        @pl.when(jnp.logical_and(j + 2 < NCH, lax.rem(j + 2, KB) == 0))
        def _():
            wait_blk(lax.rem(lax.div(j + 2, KB), 2))

        @pl.when(j + 2 < NCH)
        def _():
            start_gather(j + 2)

        nxt = lax.div(j, KB) + 1

        @pl.when(jnp.logical_and(lax.rem(j, KB) == 1,
                                 jnp.logical_and(j > KB, nxt < NBLK)))
        def _():
            start_blk(nxt, lax.rem(nxt, 2))

        return carry

    lax.fori_loop(0, NCH, body, 0)
    wait_scatter(lax.rem(NCH - 1, NBUF))
    plsc.subcore_barrier()
    pltpu.sync_copy(acc.at[sl], pacc_hbm.at[c, sl])
    if with_count:
        pltpu.sync_copy(cnt.at[sl], pcnt_hbm.at[c, sl])


_sc_agg_count = functools.partial(
    pl.kernel,
    out_type=(jax.ShapeDtypeStruct((2, N, D), jnp.float32),
              jax.ShapeDtypeStruct((2, N, 16), jnp.float32)),
    scratch_types=[
        pltpu.VMEM((2, KB, CHUNK), jnp.int32),
        pltpu.VMEM((2, KB, CHUNK), jnp.int32),
        pltpu.VMEM((NBUF, CHUNK, D), jnp.float32),
        pltpu.VMEM((CHUNK, 16), jnp.float32),
        pltpu.VMEM_SHARED((N, D), jnp.float32),
        pltpu.VMEM_SHARED((N, 16), jnp.float32),
        pltpu.SemaphoreType.DMA((2,)),
        pltpu.SemaphoreType.DMA((NBUF,)),
        pltpu.SemaphoreType.DMA((NBUF,)),
        pltpu.SemaphoreType.DMA((NBUF,)),
    ],
    mesh=_mesh,
    compiler_params=_sc_params,
)(functools.partial(_sc_agg_body, True))


_sc_agg = functools.partial(
    pl.kernel,
    out_type=jax.ShapeDtypeStruct((2, N, D), jnp.float32),
    scratch_types=[
        pltpu.VMEM((2, KB, CHUNK), jnp.int32),
        pltpu.VMEM((2, KB, CHUNK), jnp.int32),
        pltpu.VMEM((NBUF, CHUNK, D), jnp.float32),
        pltpu.VMEM_SHARED((N, D), jnp.float32),
        pltpu.SemaphoreType.DMA((2,)),
        pltpu.SemaphoreType.DMA((NBUF,)),
        pltpu.SemaphoreType.DMA((NBUF,)),
    ],
    mesh=_mesh,
    compiler_params=_sc_params,
)(functools.partial(_sc_agg_body, False))


_BN = 1000
_ROW_SPECS = [
    pl.BlockSpec((2, _BN, D), lambda i: (0, i, 0)),
    pl.BlockSpec((2, _BN, 16), lambda i: (0, i, 0)),
    pl.BlockSpec((_BN, D), lambda i: (i, 0)),
]
_W_SPEC = pl.BlockSpec((D, D), lambda i: (0, 0))
_B_SPEC = pl.BlockSpec((1, D), lambda i: (0, 0))
_OUT_SPEC = pl.BlockSpec((_BN, D), lambda i: (i, 0))
_OUT_SHAPE = jax.ShapeDtypeStruct((N, D), jnp.float32)


def _agg_of(pa_ref, pc_ref):
    asum = pa_ref[0] + pa_ref[1]
    cnt = pc_ref[0, :, 0:1] + pc_ref[1, :, 0:1]
    return asum / jnp.maximum(cnt, 1.0)


def _tc_update_body(relu, pa_ref, pc_ref, x_ref, wl_ref, b_ref, wr_ref,
                    o_ref):
    agg = _agg_of(pa_ref, pc_ref)
    h = (jnp.dot(agg, wl_ref[...], preferred_element_type=jnp.float32)
         + b_ref[...]
         + jnp.dot(x_ref[...], wr_ref[...], preferred_element_type=jnp.float32))
    o_ref[...] = jnp.maximum(h, 0.0) if relu else h


def _tc_update(pacc, pcnt, x, W_l, b_l, W_r, relu):
    return pl.pallas_call(
        functools.partial(_tc_update_body, relu), grid=(N // _BN,),
        in_specs=[*_ROW_SPECS, _W_SPEC, _B_SPEC, _W_SPEC],
        out_specs=_OUT_SPEC, out_shape=_OUT_SHAPE,
    )(pacc, pcnt, x, W_l, b_l.reshape(1, D), W_r)


def kernel(x, edge_index, W1_l, b1_l, W1_r, W2_l, b2_l, W2_r):
    ei = edge_index.astype(jnp.int32)
    src3 = ei[0].reshape(NW, NBLK, KB, CHUNK)
    dst3 = ei[1].reshape(NW, NBLK, KB, CHUNK)
    z128 = jnp.zeros((ROWS_PER_TILE, D), jnp.float32)
    z16 = jnp.zeros((ROWS_PER_TILE, 16), jnp.float32)
    ones16 = jnp.ones((CHUNK, 16), jnp.float32)

    pacc1, pcnt = _sc_agg_count(x, src3, dst3, z128, z16, ones16)
    h = _tc_update(pacc1, pcnt, x, W1_l, b1_l, W1_r, relu=True)
    pacc2 = _sc_agg(h, src3, dst3, z128)
    return _tc_update(pacc2, pcnt, h, W2_l, b2_l, W2_r, relu=False)


# speedup vs baseline: 1.0414x; 1.0222x over previous
import functools

import jax
import jax.numpy as jnp
from jax import lax
from jax.experimental import pallas as pl
from jax.experimental.pallas import tpu as pltpu
from jax.experimental.pallas import tpu_sc as plsc

N = 10000
E = 320000
D = 128
NW = 32
CHUNK = 80
NCH = E // NW // CHUNK
NBUF = 3
KB = 25
NBLK = NCH // KB
ROWS_PER_TILE = N // 16

_mesh = plsc.VectorSubcoreMesh(core_axis_name="c", subcore_axis_name="s")
_sc_params = pltpu.CompilerParams(use_tc_tiling_on_sc=False)


def _sc_agg_body(with_count, *refs):
    if with_count:
        (x_hbm, src_hbm, dst_hbm, z128_hbm, z16_hbm, ones_hbm,
         pacc_hbm, pcnt_hbm, sblk, dblk, rows, ones_v, acc, cnt,
         sem_ib, sem_g, sem_s, sem_c) = refs
    else:
        (x_hbm, src_hbm, dst_hbm, z128_hbm,
         pacc_hbm, sblk, dblk, rows, acc, sem_ib, sem_g, sem_s) = refs
        sem_c = None
    c = lax.axis_index("c")
    s = lax.axis_index("s")
    wid = s * 2 + c
    sl = pl.ds(s * ROWS_PER_TILE, ROWS_PER_TILE)
    pltpu.sync_copy(z128_hbm, acc.at[sl])
    if with_count:
        pltpu.sync_copy(z16_hbm, cnt.at[sl])
        pltpu.sync_copy(ones_hbm, ones_v)
    plsc.subcore_barrier()

    def start_blk(B, t):
        pltpu.async_copy(src_hbm.at[wid, B], sblk.at[t], sem_ib.at[t])
        pltpu.async_copy(dst_hbm.at[wid, B], dblk.at[t], sem_ib.at[t])

    def wait_blk(t):
        pltpu.make_async_copy(src_hbm.at[0, 0], sblk.at[t], sem_ib.at[t]).wait()
        pltpu.make_async_copy(dst_hbm.at[0, 0], dblk.at[t], sem_ib.at[t]).wait()

    def start_gather(j):
        t = lax.rem(lax.div(j, KB), 2)
        k = lax.rem(j, KB)
        b = lax.rem(j, NBUF)
        pltpu.async_copy(x_hbm.at[sblk.at[t, k]], rows.at[b], sem_g.at[b])

    def wait_gather(b):
        pltpu.make_async_copy(x_hbm.at[pl.ds(0, CHUNK)], rows.at[b],
                              sem_g.at[b]).wait()

    def start_scatter(j):
        t = lax.rem(lax.div(j, KB), 2)
        k = lax.rem(j, KB)
        b = lax.rem(j, NBUF)
        pltpu.async_copy(rows.at[b], acc.at[dblk.at[t, k]], sem_s.at[b],
                         add=True)
        if with_count:
            pltpu.async_copy(ones_v, cnt.at[dblk.at[t, k]], sem_c.at[b],
                             add=True)

    def wait_scatter(b):
        pltpu.make_async_copy(rows.at[b], acc.at[pl.ds(0, CHUNK)],
                              sem_s.at[b]).wait()
        if with_count:
            pltpu.make_async_copy(ones_v, cnt.at[pl.ds(0, CHUNK)],
                                  sem_c.at[b]).wait()

    start_blk(0, 0)
    start_blk(1, 1)
    wait_blk(0)
    start_gather(0)
    start_gather(1)

    def body(j, carry):
        b = lax.rem(j, NBUF)

        wait_gather(b)
        start_scatter(j)

        @pl.when(j >= 1)
        def _():
            wait_scatter(lax.rem(j - 1, NBUF))

        @pl.when(jnp.logical_and(j + 2 < NCH, lax.rem(j + 2, KB) == 0))
        def _():
            wait_blk(lax.rem(lax.div(j + 2, KB), 2))

        @pl.when(j + 2 < NCH)
        def _():
            start_gather(j + 2)

        nxt = lax.div(j, KB) + 1

        @pl.when(jnp.logical_and(lax.rem(j, KB) == 1,
                                 jnp.logical_and(j > KB, nxt < NBLK)))
        def _():
            start_blk(nxt, lax.rem(nxt, 2))

        return carry

    lax.fori_loop(0, NCH, body, 0)
    wait_scatter(lax.rem(NCH - 1, NBUF))
    plsc.subcore_barrier()
    pltpu.sync_copy(acc.at[sl], pacc_hbm.at[c, sl])
    if with_count:
        pltpu.sync_copy(cnt.at[sl], pcnt_hbm.at[c, sl])


_sc_agg_count = functools.partial(
    pl.kernel,
    out_type=(jax.ShapeDtypeStruct((2, N, D), jnp.float32),
              jax.ShapeDtypeStruct((2, N, 16), jnp.float32)),
    scratch_types=[
        pltpu.VMEM((2, KB, CHUNK), jnp.int32),
        pltpu.VMEM((2, KB, CHUNK), jnp.int32),
        pltpu.VMEM((NBUF, CHUNK, D), jnp.float32),
        pltpu.VMEM((CHUNK, 16), jnp.float32),
        pltpu.VMEM_SHARED((N, D), jnp.float32),
        pltpu.VMEM_SHARED((N, 16), jnp.float32),
        pltpu.SemaphoreType.DMA((2,)),
        pltpu.SemaphoreType.DMA((NBUF,)),
        pltpu.SemaphoreType.DMA((NBUF,)),
        pltpu.SemaphoreType.DMA((NBUF,)),
    ],
    mesh=_mesh,
    compiler_params=_sc_params,
)(functools.partial(_sc_agg_body, True))


_sc_agg = functools.partial(
    pl.kernel,
    out_type=jax.ShapeDtypeStruct((2, N, D), jnp.float32),
    scratch_types=[
        pltpu.VMEM((2, KB, CHUNK), jnp.int32),
        pltpu.VMEM((2, KB, CHUNK), jnp.int32),
        pltpu.VMEM((NBUF, CHUNK, D), jnp.float32),
        pltpu.VMEM_SHARED((N, D), jnp.float32),
        pltpu.SemaphoreType.DMA((2,)),
        pltpu.SemaphoreType.DMA((NBUF,)),
        pltpu.SemaphoreType.DMA((NBUF,)),
    ],
    mesh=_mesh,
    compiler_params=_sc_params,
)(functools.partial(_sc_agg_body, False))


_BN = 2000
_ROW_SPECS = [
    pl.BlockSpec((2, _BN, D), lambda i: (0, i, 0)),
    pl.BlockSpec((2, _BN, 16), lambda i: (0, i, 0)),
    pl.BlockSpec((_BN, D), lambda i: (i, 0)),
]
_W_SPEC = pl.BlockSpec((D, D), lambda i: (0, 0))
_B_SPEC = pl.BlockSpec((1, D), lambda i: (0, 0))
_OUT_SPEC = pl.BlockSpec((_BN, D), lambda i: (i, 0))
_OUT_SHAPE = jax.ShapeDtypeStruct((N, D), jnp.float32)


def _agg_of(pa_ref, pc_ref):
    asum = pa_ref[0] + pa_ref[1]
    cnt = pc_ref[0, :, 0:1] + pc_ref[1, :, 0:1]
    return asum / jnp.maximum(cnt, 1.0)


def _tc_update_body(relu, pa_ref, pc_ref, x_ref, wl_ref, b_ref, wr_ref,
                    o_ref):
    agg = _agg_of(pa_ref, pc_ref)
    h = (jnp.dot(agg, wl_ref[...], preferred_element_type=jnp.float32)
         + b_ref[...]
         + jnp.dot(x_ref[...], wr_ref[...], preferred_element_type=jnp.float32))
    o_ref[...] = jnp.maximum(h, 0.0) if relu else h


def _tc_update(pacc, pcnt, x, W_l, b_l, W_r, relu):
    return pl.pallas_call(
        functools.partial(_tc_update_body, relu), grid=(N // _BN,),
        in_specs=[*_ROW_SPECS, _W_SPEC, _B_SPEC, _W_SPEC],
        out_specs=_OUT_SPEC, out_shape=_OUT_SHAPE,
    )(pacc, pcnt, x, W_l, b_l.reshape(1, D), W_r)


def kernel(x, edge_index, W1_l, b1_l, W1_r, W2_l, b2_l, W2_r):
    ei = edge_index.astype(jnp.int32)
    src3 = ei[0].reshape(NW, NBLK, KB, CHUNK)
    dst3 = ei[1].reshape(NW, NBLK, KB, CHUNK)
    z128 = jnp.zeros((ROWS_PER_TILE, D), jnp.float32)
    z16 = jnp.zeros((ROWS_PER_TILE, 16), jnp.float32)
    ones16 = jnp.ones((CHUNK, 16), jnp.float32)

    pacc1, pcnt = _sc_agg_count(x, src3, dst3, z128, z16, ones16)
    h = _tc_update(pacc1, pcnt, x, W1_l, b1_l, W1_r, relu=True)
    pacc2 = _sc_agg(h, src3, dst3, z128)
    return _tc_update(pacc2, pcnt, h, W2_l, b2_l, W2_r, relu=False)


# speedup vs baseline: 1.0482x; 1.0065x over previous
import functools

import jax
import jax.numpy as jnp
from jax import lax
from jax.experimental import pallas as pl
from jax.experimental.pallas import tpu as pltpu
from jax.experimental.pallas import tpu_sc as plsc

N = 10000
E = 320000
D = 128
NW = 32
CHUNK = 80
NCH = E // NW // CHUNK
NBUF = 3
KB = 25
NBLK = NCH // KB
ROWS_PER_TILE = N // 16

_mesh = plsc.VectorSubcoreMesh(core_axis_name="c", subcore_axis_name="s")
_sc_params = pltpu.CompilerParams(use_tc_tiling_on_sc=False)


def _sc_agg_body(with_count, *refs):
    if with_count:
        (x_hbm, src_hbm, dst_hbm, z128_hbm, z16_hbm, ones_hbm,
         pacc_hbm, pcnt_hbm, sblk, dblk, rows, ones_v, acc, cnt,
         sem_ib, sem_g, sem_s, sem_c) = refs
    else:
        (x_hbm, src_hbm, dst_hbm, z128_hbm,
         pacc_hbm, sblk, dblk, rows, acc, sem_ib, sem_g, sem_s) = refs
        sem_c = None
    c = lax.axis_index("c")
    s = lax.axis_index("s")
    wid = s * 2 + c
    sl = pl.ds(s * ROWS_PER_TILE, ROWS_PER_TILE)

    def start_blk(B, t):
        pltpu.async_copy(src_hbm.at[wid, B], sblk.at[t], sem_ib.at[t])
        pltpu.async_copy(dst_hbm.at[wid, B], dblk.at[t], sem_ib.at[t])

    def wait_blk(t):
        pltpu.make_async_copy(src_hbm.at[0, 0], sblk.at[t], sem_ib.at[t]).wait()
        pltpu.make_async_copy(dst_hbm.at[0, 0], dblk.at[t], sem_ib.at[t]).wait()

    def start_gather(j):
        t = lax.rem(lax.div(j, KB), 2)
        k = lax.rem(j, KB)
        b = lax.rem(j, NBUF)
        pltpu.async_copy(x_hbm.at[sblk.at[t, k]], rows.at[b], sem_g.at[b])

    def wait_gather(b):
        pltpu.make_async_copy(x_hbm.at[pl.ds(0, CHUNK)], rows.at[b],
                              sem_g.at[b]).wait()

    def start_scatter(j):
        t = lax.rem(lax.div(j, KB), 2)
        k = lax.rem(j, KB)
        b = lax.rem(j, NBUF)
        pltpu.async_copy(rows.at[b], acc.at[dblk.at[t, k]], sem_s.at[b],
                         add=True)
        if with_count:
            pltpu.async_copy(ones_v, cnt.at[dblk.at[t, k]], sem_c.at[b],
                             add=True)

    def wait_scatter(b):
        pltpu.make_async_copy(rows.at[b], acc.at[pl.ds(0, CHUNK)],
                              sem_s.at[b]).wait()
        if with_count:
            pltpu.make_async_copy(ones_v, cnt.at[pl.ds(0, CHUNK)],
                                  sem_c.at[b]).wait()

    start_blk(0, 0)
    start_blk(1, 1)
    pltpu.sync_copy(z128_hbm, acc.at[sl])
    if with_count:
        pltpu.sync_copy(z16_hbm, cnt.at[sl])
        pltpu.sync_copy(ones_hbm, ones_v)
    wait_blk(0)
    start_gather(0)
    start_gather(1)
    plsc.subcore_barrier()

    def body(j, carry):
        b = lax.rem(j, NBUF)

        wait_gather(b)
        start_scatter(j)

        @pl.when(j >= 1)
        def _():
            wait_scatter(lax.rem(j - 1, NBUF))

        @pl.when(jnp.logical_and(j + 2 < NCH, lax.rem(j + 2, KB) == 0))
        def _():
            wait_blk(lax.rem(lax.div(j + 2, KB), 2))

        @pl.when(j + 2 < NCH)
        def _():
            start_gather(j + 2)

        nxt = lax.div(j, KB) + 1

        @pl.when(jnp.logical_and(lax.rem(j, KB) == 1,
                                 jnp.logical_and(j > KB, nxt < NBLK)))
        def _():
            start_blk(nxt, lax.rem(nxt, 2))

        return carry

    lax.fori_loop(0, NCH, body, 0)
    wait_scatter(lax.rem(NCH - 1, NBUF))
    plsc.subcore_barrier()
    pltpu.sync_copy(acc.at[sl], pacc_hbm.at[c, sl])
    if with_count:
        pltpu.sync_copy(cnt.at[sl], pcnt_hbm.at[c, sl])


_sc_agg_count = functools.partial(
    pl.kernel,
    out_type=(jax.ShapeDtypeStruct((2, N, D), jnp.float32),
              jax.ShapeDtypeStruct((2, N, 16), jnp.float32)),
    scratch_types=[
        pltpu.VMEM((2, KB, CHUNK), jnp.int32),
        pltpu.VMEM((2, KB, CHUNK), jnp.int32),
        pltpu.VMEM((NBUF, CHUNK, D), jnp.float32),
        pltpu.VMEM((CHUNK, 16), jnp.float32),
        pltpu.VMEM_SHARED((N, D), jnp.float32),
        pltpu.VMEM_SHARED((N, 16), jnp.float32),
        pltpu.SemaphoreType.DMA((2,)),
        pltpu.SemaphoreType.DMA((NBUF,)),
        pltpu.SemaphoreType.DMA((NBUF,)),
        pltpu.SemaphoreType.DMA((NBUF,)),
    ],
    mesh=_mesh,
    compiler_params=_sc_params,
)(functools.partial(_sc_agg_body, True))


_sc_agg = functools.partial(
    pl.kernel,
    out_type=jax.ShapeDtypeStruct((2, N, D), jnp.float32),
    scratch_types=[
        pltpu.VMEM((2, KB, CHUNK), jnp.int32),
        pltpu.VMEM((2, KB, CHUNK), jnp.int32),
        pltpu.VMEM((NBUF, CHUNK, D), jnp.float32),
        pltpu.VMEM_SHARED((N, D), jnp.float32),
        pltpu.SemaphoreType.DMA((2,)),
        pltpu.SemaphoreType.DMA((NBUF,)),
        pltpu.SemaphoreType.DMA((NBUF,)),
    ],
    mesh=_mesh,
    compiler_params=_sc_params,
)(functools.partial(_sc_agg_body, False))


_BN = 2000
_ROW_SPECS = [
    pl.BlockSpec((2, _BN, D), lambda i: (0, i, 0)),
    pl.BlockSpec((2, _BN, 16), lambda i: (0, i, 0)),
    pl.BlockSpec((_BN, D), lambda i: (i, 0)),
]
_W_SPEC = pl.BlockSpec((D, D), lambda i: (0, 0))
_B_SPEC = pl.BlockSpec((1, D), lambda i: (0, 0))
_OUT_SPEC = pl.BlockSpec((_BN, D), lambda i: (i, 0))
_OUT_SHAPE = jax.ShapeDtypeStruct((N, D), jnp.float32)


def _agg_of(pa_ref, pc_ref):
    asum = pa_ref[0] + pa_ref[1]
    cnt = pc_ref[0, :, 0:1] + pc_ref[1, :, 0:1]
    return asum / jnp.maximum(cnt, 1.0)


def _tc_update_body(relu, pa_ref, pc_ref, x_ref, wl_ref, b_ref, wr_ref,
                    o_ref):
    agg = _agg_of(pa_ref, pc_ref)
    h = (jnp.dot(agg, wl_ref[...], preferred_element_type=jnp.float32)
         + b_ref[...]
         + jnp.dot(x_ref[...], wr_ref[...], preferred_element_type=jnp.float32))
    o_ref[...] = jnp.maximum(h, 0.0) if relu else h


def _tc_update(pacc, pcnt, x, W_l, b_l, W_r, relu):
    return pl.pallas_call(
        functools.partial(_tc_update_body, relu), grid=(N // _BN,),
        in_specs=[*_ROW_SPECS, _W_SPEC, _B_SPEC, _W_SPEC],
        out_specs=_OUT_SPEC, out_shape=_OUT_SHAPE,
    )(pacc, pcnt, x, W_l, b_l.reshape(1, D), W_r)


def kernel(x, edge_index, W1_l, b1_l, W1_r, W2_l, b2_l, W2_r):
    ei = edge_index.astype(jnp.int32)
    src3 = ei[0].reshape(NW, NBLK, KB, CHUNK)
    dst3 = ei[1].reshape(NW, NBLK, KB, CHUNK)
    z128 = jnp.zeros((ROWS_PER_TILE, D), jnp.float32)
    z16 = jnp.zeros((ROWS_PER_TILE, 16), jnp.float32)
    ones16 = jnp.ones((CHUNK, 16), jnp.float32)

    pacc1, pcnt = _sc_agg_count(x, src3, dst3, z128, z16, ones16)
    h = _tc_update(pacc1, pcnt, x, W1_l, b1_l, W1_r, relu=True)
    pacc2 = _sc_agg(h, src3, dst3, z128)
    return _tc_update(pacc2, pcnt, h, W2_l, b2_l, W2_r, relu=False)
